# baseline (speedup 1.0000x reference)
import functools

import jax
import jax.numpy as jnp
from jax import lax
from jax.experimental import pallas as pl
from jax.experimental.pallas import tpu as pltpu
from jax.experimental.pallas import tpu_sc as plsc

B = 64
N = 32768
NC = 2
NS = 16
L = 16
NW = NC * NS
ROWS_PER_W = B // NW
NVEC = N // L
U = 8

_NEG = -3.0e38


def _splat_f(x):
    return jnp.full((L,), x, jnp.float32)


CH = 4
CHN = N // CH


def _row_tau(row_v, cand_v, in_copies):
    ones = jnp.full((L,), 1, jnp.int32)
    zeros = jnp.full((L,), 0, jnp.int32)
    iota = lax.iota(jnp.int32, L)

    RU = N // U

    def sweep_body(i, carry):
        thp, acc, idxs = carry
        vs = [row_v[pl.ds((i * U + u) * L, L)] for u in range(U)]
        new_idxs = []
        for u in range(U):
            msk = vs[u] >= thp
            plsc.store_scatter(cand_v, [idxs[u]], vs[u], mask=msk)
            new_idxs.append(
                idxs[u] + jnp.where(msk, jnp.int32(L), jnp.int32(0)))
        bmax = vs
        while len(bmax) > 1:
            bmax = [jnp.maximum(a, b) for a, b in zip(bmax[::2], bmax[1::2])]
        acc = jnp.maximum(acc, bmax[0])
        thp = jnp.maximum(thp, bmax[0] - jnp.float32(1.0))
        return thp, acc, tuple(new_idxs)

    idxs0 = tuple(jnp.full((L,), u * RU, jnp.int32) + iota for u in range(U))
    in_copies[0].wait()
    svs = [row_v[pl.ds(k * 16 * L, L)] for k in range(32)]
    while len(svs) > 1:
        svs = [jnp.maximum(a, b) for a, b in zip(svs[::2], svs[1::2])]
    smax = svs[0]
    carry = (smax - jnp.float32(1.0), smax, idxs0)
    steps = NVEC // U // CH
    for c in range(CH):
        if c:
            in_copies[c].wait()
        carry = lax.fori_loop(c * steps, (c + 1) * steps, sweep_body, carry)
    _, acc, idxs_v = carry
    cnts = [jnp.right_shift(idxs_v[u] - iota, 4) - jnp.int32(u * (RU >> 4))
            for u in range(U)]

    m = jnp.max(acc)
    cmax = cnts
    while len(cmax) > 1:
        cmax = [jnp.maximum(a, b) for a, b in zip(cmax[::2], cmax[1::2])]
    nv = jnp.max(cmax[0])

    def tau_from(sel_fn):
        def body(j, acc2):
            s, c = acc2
            for u in range(U):
                v = cand_v[pl.ds(u * RU + j * L, L)]
                sel = jnp.logical_and(cnts[u] > j, sel_fn(v))
                s = s + jnp.where(sel, v, jnp.float32(0.0))
                c = c + jnp.where(sel, ones, zeros)
            return s, c

        s, c = lax.fori_loop(0, nv, body, (_splat_f(0.0), zeros))
        cs = jnp.sum(c)
        sv = _splat_f(jnp.sum(s))
        cv = jnp.full((L,), cs).astype(jnp.float32)
        return (sv - jnp.float32(1.0)) / cv, cs

    t0, _ = tau_from(lambda v: v >= m)

    def fx_body(i, st):
        t, _, cnow = st
        t2, c = tau_from(lambda v: v > t)
        return (t2, cnow, c)

    t6, cp6, cn6 = lax.fori_loop(
        0, 6, fx_body, (t0, jnp.int32(-1), jnp.int32(-2)))

    def w_cond(st):
        _, cprev, cnow, it = st
        return jnp.logical_and(cnow != cprev, it < jnp.int32(128))

    def w_body(st):
        t, _, cnow, it = st
        t2, c = tau_from(lambda v: v > t)
        return (t2, cnow, c, it + jnp.int32(1))

    tau, _, _, _ = lax.while_loop(
        w_cond, w_body, (t6, cp6, cn6, jnp.int32(0)))
    return tau


def _row_out(row_v, tau, out_hbm_row, out_sem):
    def out_body(i, _):
        for u in range(U):
            sl = pl.ds((i * U + u) * L, L)
            row_v[sl] = jnp.maximum(row_v[sl] - tau, jnp.float32(0.0))
        return 0

    steps = NVEC // U // CH
    copies = []
    for c in range(CH):
        lax.fori_loop(c * steps, (c + 1) * steps, out_body, 0)
        cp = pltpu.make_async_copy(
            row_v.at[pl.ds(c * CHN, CHN)],
            out_hbm_row.at[pl.ds(c * CHN, CHN)], out_sem)
        cp.start()
        copies.append(cp)
    return copies


@functools.partial(
    pl.kernel,
    out_type=jax.ShapeDtypeStruct((B, N), jnp.float32),
    mesh=plsc.VectorSubcoreMesh(core_axis_name="c", subcore_axis_name="s"),
    compiler_params=pltpu.CompilerParams(needs_layout_passes=False),
    scratch_types=[
        pltpu.VMEM((N,), jnp.float32),
        pltpu.VMEM((N,), jnp.float32),
        pltpu.VMEM((N,), jnp.float32),
        pltpu.SemaphoreType.DMA,
        pltpu.SemaphoreType.DMA,
        pltpu.SemaphoreType.DMA,
        pltpu.SemaphoreType.DMA,
    ],
)
def _sparsemax_sc(z_hbm, out_hbm, row0_v, row1_v, cand_v,
                  in0_sem, in1_sem, out0_sem, out1_sem):
    wid = lax.axis_index("s") * NC + lax.axis_index("c")
    r0 = wid * ROWS_PER_W
    r1 = r0 + 1
    in0 = [pltpu.make_async_copy(z_hbm.at[r0, pl.ds(c * CHN, CHN)],
                                 row0_v.at[pl.ds(c * CHN, CHN)], in0_sem)
           for c in range(CH)]
    in1 = [pltpu.make_async_copy(z_hbm.at[r1, pl.ds(c * CHN, CHN)],
                                 row1_v.at[pl.ds(c * CHN, CHN)], in1_sem)
           for c in range(CH)]
    for cp in in0 + in1:
        cp.start()
    tau0 = _row_tau(row0_v, cand_v, in0)
    out0 = _row_out(row0_v, tau0, out_hbm.at[r0], out0_sem)
    tau1 = _row_tau(row1_v, cand_v, in1)
    out1 = _row_out(row1_v, tau1, out_hbm.at[r1], out1_sem)
    for cp in out0 + out1:
        cp.wait()


def kernel(z):
    assert z.shape == (B, N) and z.dtype == jnp.float32
    return _sparsemax_sc(z)


# speedup vs baseline: 1.0241x; 1.0241x over previous
import functools

import jax
import jax.numpy as jnp
from jax import lax
from jax.experimental import pallas as pl
from jax.experimental.pallas import tpu as pltpu
from jax.experimental.pallas import tpu_sc as plsc

B = 64
N = 32768
NC = 2
NS = 16
L = 16
NW = NC * NS
ROWS_PER_W = B // NW
NVEC = N // L
U = 8

_NEG = -3.0e38


def _splat_f(x):
    return jnp.full((L,), x, jnp.float32)


CH = 4
CHN = N // CH


def _row_tau(row_v, cand_v, in_copies):
    ones = jnp.full((L,), 1, jnp.int32)
    zeros = jnp.full((L,), 0, jnp.int32)
    iota = lax.iota(jnp.int32, L)

    RU = N // U

    def sweep_body(i, carry):
        thp, acc, idxs = carry
        vs = [row_v[pl.ds((i * U + u) * L, L)] for u in range(U)]
        new_idxs = []
        for u in range(U):
            msk = vs[u] >= thp
            plsc.store_scatter(cand_v, [idxs[u]], vs[u], mask=msk)
            new_idxs.append(
                idxs[u] + jnp.where(msk, jnp.int32(L), jnp.int32(0)))
        bmax = vs
        while len(bmax) > 1:
            bmax = [jnp.maximum(a, b) for a, b in zip(bmax[::2], bmax[1::2])]
        acc = jnp.maximum(acc, bmax[0])
        thp = jnp.maximum(thp, bmax[0] - jnp.float32(1.0))
        return thp, acc, tuple(new_idxs)

    idxs0 = tuple(jnp.full((L,), u * RU, jnp.int32) + iota for u in range(U))
    in_copies[0].wait()
    svs = [row_v[pl.ds(k * 16 * L, L)] for k in range(32)]
    while len(svs) > 1:
        svs = [jnp.maximum(a, b) for a, b in zip(svs[::2], svs[1::2])]
    smax = svs[0]
    carry = (smax - jnp.float32(1.0), smax, idxs0)
    steps = NVEC // U // CH
    for c in range(CH):
        if c:
            in_copies[c].wait()
        carry = lax.fori_loop(c * steps, (c + 1) * steps, sweep_body, carry)
    _, acc, idxs_v = carry
    cnts = [jnp.right_shift(idxs_v[u] - iota, 4) - jnp.int32(u * (RU >> 4))
            for u in range(U)]

    m = jnp.max(acc)
    cmax = cnts
    while len(cmax) > 1:
        cmax = [jnp.maximum(a, b) for a, b in zip(cmax[::2], cmax[1::2])]
    nv = jnp.max(cmax[0])

    def tau_from(sel_fn):
        def body(j, acc2):
            s, c = acc2
            for u in range(U):
                v = cand_v[pl.ds(u * RU + j * L, L)]
                sel = jnp.logical_and(cnts[u] > j, sel_fn(v))
                s = s + jnp.where(sel, v, jnp.float32(0.0))
                c = c + jnp.where(sel, ones, zeros)
            return s, c

        s, c = lax.fori_loop(0, nv, body, (_splat_f(0.0), zeros))
        cs = jnp.sum(c)
        sv = _splat_f(jnp.sum(s))
        cv = jnp.full((L,), cs).astype(jnp.float32)
        return (sv - jnp.float32(1.0)) / cv, cs

    t0, _ = tau_from(lambda v: v >= m)

    def w_cond(st):
        _, cprev, cnow, it = st
        return jnp.logical_and(cnow != cprev, it < jnp.int32(128))

    def w_body(st):
        t, _, cnow, it = st
        t2, c = tau_from(lambda v: v > t)
        return (t2, cnow, c, it + jnp.int32(1))

    tau, _, _, _ = lax.while_loop(
        w_cond, w_body, (t0, jnp.int32(-1), jnp.int32(-2), jnp.int32(0)))
    return tau


def _row_out(row_v, tau, out_hbm_row, out_sem):
    def out_body(i, _):
        for u in range(U):
            sl = pl.ds((i * U + u) * L, L)
            row_v[sl] = jnp.maximum(row_v[sl] - tau, jnp.float32(0.0))
        return 0

    steps = NVEC // U // CH
    copies = []
    for c in range(CH):
        lax.fori_loop(c * steps, (c + 1) * steps, out_body, 0)
        cp = pltpu.make_async_copy(
            row_v.at[pl.ds(c * CHN, CHN)],
            out_hbm_row.at[pl.ds(c * CHN, CHN)], out_sem)
        cp.start()
        copies.append(cp)
    return copies


@functools.partial(
    pl.kernel,
    out_type=jax.ShapeDtypeStruct((B, N), jnp.float32),
    mesh=plsc.VectorSubcoreMesh(core_axis_name="c", subcore_axis_name="s"),
    compiler_params=pltpu.CompilerParams(needs_layout_passes=False),
    scratch_types=[
        pltpu.VMEM((N,), jnp.float32),
        pltpu.VMEM((N,), jnp.float32),
        pltpu.VMEM((N,), jnp.float32),
        pltpu.SemaphoreType.DMA,
        pltpu.SemaphoreType.DMA,
        pltpu.SemaphoreType.DMA,
        pltpu.SemaphoreType.DMA,
    ],
)
def _sparsemax_sc(z_hbm, out_hbm, row0_v, row1_v, cand_v,
                  in0_sem, in1_sem, out0_sem, out1_sem):
    wid = lax.axis_index("s") * NC + lax.axis_index("c")
    r0 = wid * ROWS_PER_W
    r1 = r0 + 1
    in0 = [pltpu.make_async_copy(z_hbm.at[r0, pl.ds(c * CHN, CHN)],
                                 row0_v.at[pl.ds(c * CHN, CHN)], in0_sem)
           for c in range(CH)]
    in1 = [pltpu.make_async_copy(z_hbm.at[r1, pl.ds(c * CHN, CHN)],
                                 row1_v.at[pl.ds(c * CHN, CHN)], in1_sem)
           for c in range(CH)]
    for cp in in0 + in1:
        cp.start()
    tau0 = _row_tau(row0_v, cand_v, in0)
    out0 = _row_out(row0_v, tau0, out_hbm.at[r0], out0_sem)
    tau1 = _row_tau(row1_v, cand_v, in1)
    out1 = _row_out(row1_v, tau1, out_hbm.at[r1], out1_sem)
    for cp in out0 + out1:
        cp.wait()


def kernel(z):
    assert z.shape == (B, N) and z.dtype == jnp.float32
    return _sparsemax_sc(z)
